# speedup vs baseline: 2.4774x; 2.4774x over previous
import jax
import jax.numpy as jnp
from jax import lax
from jax.experimental import pallas as pl
from jax.experimental.pallas import tpu as pltpu
from jax.experimental.pallas import tpu_sc as plsc

N = 10000
F = 128
E = 320000

NP = 10240
NW = 32
C = 128
EPW = NP
CH = EPW // C
EP = NW * EPW
RPT = NP // 16
BLK = 512


def _dense_body(p0_ref, p1_ref, f_ref, a_ref, b_ref, bias_ref, w_ref, o_ref):
    h = p0_ref[...] + p1_ref[...]
    hid = jnp.maximum(
        jnp.dot(h, a_ref[...], preferred_element_type=jnp.float32)
        + jnp.dot(f_ref[...], b_ref[...], preferred_element_type=jnp.float32)
        + bias_ref[...],
        0.0,
    )
    o_ref[...] = jnp.dot(hid, w_ref[...], preferred_element_type=jnp.float32)


def _dense_last_body(p0_ref, p1_ref, f_ref, a_ref, b_ref, bias_ref, o_ref):
    h = p0_ref[...] + p1_ref[...]
    o_ref[...] = jnp.maximum(
        jnp.dot(h, a_ref[...], preferred_element_type=jnp.float32)
        + jnp.dot(f_ref[...], b_ref[...], preferred_element_type=jnp.float32)
        + bias_ref[...],
        0.0,
    )


def _mm_body(x_ref, w_ref, o_ref):
    o_ref[...] = jnp.dot(x_ref[...], w_ref[...], preferred_element_type=jnp.float32)


_row_spec = pl.BlockSpec((BLK, F), lambda i: (i, 0))
_w_spec = pl.BlockSpec((F, F), lambda i: (0, 0))
_bias_spec = pl.BlockSpec((1, F), lambda i: (0, 0))
_p1_spec = pl.BlockSpec((BLK, F), lambda i: (i + NP // BLK, 0))

_GRID = (NP // BLK,)
_OUT = jax.ShapeDtypeStruct((NP, F), jnp.float32)

_mm = pl.pallas_call(
    _mm_body,
    grid=_GRID,
    in_specs=[_row_spec, _w_spec],
    out_specs=_row_spec,
    out_shape=_OUT,
)

_dense = pl.pallas_call(
    _dense_body,
    grid=_GRID,
    in_specs=[_row_spec, _p1_spec, _row_spec, _w_spec, _w_spec, _bias_spec, _w_spec],
    out_specs=_row_spec,
    out_shape=_OUT,
)

_dense_last = pl.pallas_call(
    _dense_last_body,
    grid=_GRID,
    in_specs=[_row_spec, _p1_spec, _row_spec, _w_spec, _w_spec, _bias_spec],
    out_specs=_row_spec,
    out_shape=_OUT,
)


def _spmm_body(y_hbm, src_hbm, dst_hbm, zeros_hbm, out_hbm,
               src_v, dst_v, buf, acc, gsem, ssem):
    cid = lax.axis_index("c")
    sid = lax.axis_index("s")
    w = cid * 16 + sid
    pltpu.sync_copy(src_hbm.at[w], src_v)
    pltpu.sync_copy(dst_hbm.at[w], dst_v)
    pltpu.sync_copy(zeros_hbm, acc.at[pl.ds(sid * RPT, RPT)])
    plsc.subcore_barrier()

    def body(j, carry):
        pltpu.async_copy(y_hbm.at[src_v.at[j]], buf, gsem).wait()
        pltpu.async_copy(buf, acc.at[dst_v.at[j]], ssem, add=True).wait()
        return carry

    lax.fori_loop(0, CH, body, 0)
    plsc.subcore_barrier()
    pltpu.sync_copy(acc.at[pl.ds(sid * RPT, RPT)],
                    out_hbm.at[pl.ds(cid * NP + sid * RPT, RPT)])


def _spmm(y, src_p, dst_p, zeros_blk):
    fn = pl.kernel(
        _spmm_body,
        out_type=jax.ShapeDtypeStruct((2 * NP, F), jnp.float32),
        mesh=plsc.VectorSubcoreMesh(core_axis_name="c", subcore_axis_name="s"),
        scratch_types=[
            pltpu.VMEM((CH, C), jnp.int32),
            pltpu.VMEM((CH, C), jnp.int32),
            pltpu.VMEM((C, F), jnp.float32),
            pltpu.VMEM_SHARED((NP, F), jnp.float32),
            pltpu.SemaphoreType.DMA,
            pltpu.SemaphoreType.DMA,
        ],
    )
    return fn(y, src_p, dst_p, zeros_blk)


def kernel(features, edge_index, W0, W1, W2, ThW, Thb, Th1W, Th1b, Th2W, Th2b):
    f_pad = jnp.pad(features, ((0, NP - N), (0, 0)))
    dst = edge_index[0]
    src = edge_index[1]
    pad_e = EP - E
    src_p = jnp.concatenate([src, jnp.zeros((pad_e,), jnp.int32)]).reshape(NW, CH, C)
    dst_p = jnp.concatenate([dst, jnp.full((pad_e,), N, jnp.int32)]).reshape(NW, CH, C)
    zeros_blk = jnp.zeros((RPT, F), jnp.float32)

    A0, B0 = ThW[:F], ThW[F:]
    A1, B1 = Th1W[:F], Th1W[F:]
    A2, B2 = Th2W[:F], Th2W[F:]
    b0 = Thb.reshape(1, F)
    b1 = Th1b.reshape(1, F)
    b2 = Th2b.reshape(1, F)

    y = _mm(f_pad, W0)
    p = _spmm(y, src_p, dst_p, zeros_blk)
    y = _dense(p, p, f_pad, A0, B0, b0, W1)
    p = _spmm(y, src_p, dst_p, zeros_blk)
    y = _dense(p, p, f_pad, A1, B1, b1, W2)
    p = _spmm(y, src_p, dst_p, zeros_blk)
    emb = _dense_last(p, p, f_pad, A2, B2, b2)
    return emb[:N]


# speedup vs baseline: 8.9090x; 3.5961x over previous
import jax
import jax.numpy as jnp
from jax import lax
from jax.experimental import pallas as pl
from jax.experimental.pallas import tpu as pltpu
from jax.experimental.pallas import tpu_sc as plsc

N = 10000
F = 128
E = 320000

NP = 10240
NW = 32
C = 128
NSEG = 2
SEGR = 40
SEGCH = SEGR
EPW = NSEG * SEGR * C
EP = NW * EPW
RPT = NP // 16
BLK = 512


def _dense_body(p0_ref, p1_ref, f_ref, a_ref, b_ref, bias_ref, w_ref, o_ref):
    h = p0_ref[...] + p1_ref[...]
    hid = jnp.maximum(
        jnp.dot(h, a_ref[...], preferred_element_type=jnp.float32)
        + jnp.dot(f_ref[...], b_ref[...], preferred_element_type=jnp.float32)
        + bias_ref[...],
        0.0,
    )
    o_ref[...] = jnp.dot(hid, w_ref[...], preferred_element_type=jnp.float32)


def _dense_last_body(p0_ref, p1_ref, f_ref, a_ref, b_ref, bias_ref, o_ref):
    h = p0_ref[...] + p1_ref[...]
    o_ref[...] = jnp.maximum(
        jnp.dot(h, a_ref[...], preferred_element_type=jnp.float32)
        + jnp.dot(f_ref[...], b_ref[...], preferred_element_type=jnp.float32)
        + bias_ref[...],
        0.0,
    )


def _mm_body(x_ref, w_ref, o_ref):
    o_ref[...] = jnp.dot(x_ref[...], w_ref[...], preferred_element_type=jnp.float32)


_row_spec = pl.BlockSpec((BLK, F), lambda i: (i, 0))
_w_spec = pl.BlockSpec((F, F), lambda i: (0, 0))
_bias_spec = pl.BlockSpec((1, F), lambda i: (0, 0))
_p1_spec = pl.BlockSpec((BLK, F), lambda i: (i + NP // BLK, 0))

_GRID = (NP // BLK,)
_OUT = jax.ShapeDtypeStruct((NP, F), jnp.float32)

_mm = pl.pallas_call(
    _mm_body,
    grid=_GRID,
    in_specs=[_row_spec, _w_spec],
    out_specs=_row_spec,
    out_shape=_OUT,
)

_dense = pl.pallas_call(
    _dense_body,
    grid=_GRID,
    in_specs=[_row_spec, _p1_spec, _row_spec, _w_spec, _w_spec, _bias_spec, _w_spec],
    out_specs=_row_spec,
    out_shape=_OUT,
)

_dense_last = pl.pallas_call(
    _dense_last_body,
    grid=_GRID,
    in_specs=[_row_spec, _p1_spec, _row_spec, _w_spec, _w_spec, _bias_spec],
    out_specs=_row_spec,
    out_shape=_OUT,
)


def _spmm_body(y_hbm, src_hbm, dst_hbm, zeros_hbm, out_hbm,
               src_v, dst_v, b0, b1, acc, g0, g1, s0, s1):
    cid = lax.axis_index("c")
    sid = lax.axis_index("s")
    w = cid * 16 + sid
    pltpu.sync_copy(zeros_hbm, acc.at[pl.ds(sid * RPT, RPT)])
    plsc.subcore_barrier()

    def gath(c, buf, sem):
        return pltpu.async_copy(y_hbm.at[src_v.at[c]], buf, sem)

    def scat(c, buf, sem):
        return pltpu.async_copy(buf, acc.at[dst_v.at[c]], sem, add=True)

    def seg_body(seg, carry):
        pltpu.sync_copy(src_hbm.at[w, seg], src_v)
        pltpu.sync_copy(dst_hbm.at[w, seg], dst_v)

        def body(t, carry):
            a = 2 * t
            b = a + 1
            ga = gath(a, b0, g0)
            gb = gath(b, b1, g1)
            ga.wait()
            sa = scat(a, b0, s0)
            gb.wait()
            sb = scat(b, b1, s1)
            sa.wait()
            sb.wait()
            return carry

        lax.fori_loop(0, SEGCH // 2, body, 0)
        return carry

    lax.fori_loop(0, NSEG, seg_body, 0)
    plsc.subcore_barrier()
    pltpu.sync_copy(acc.at[pl.ds(sid * RPT, RPT)],
                    out_hbm.at[pl.ds(cid * NP + sid * RPT, RPT)])


def _spmm(y, src_p, dst_p, zeros_blk):
    fn = pl.kernel(
        _spmm_body,
        out_type=jax.ShapeDtypeStruct((2 * NP, F), jnp.float32),
        mesh=plsc.VectorSubcoreMesh(core_axis_name="c", subcore_axis_name="s"),
        scratch_types=[
            pltpu.VMEM((SEGCH, C), jnp.int32),
            pltpu.VMEM((SEGCH, C), jnp.int32),
            pltpu.VMEM((C, F), jnp.float32),
            pltpu.VMEM((C, F), jnp.float32),
            pltpu.VMEM_SHARED((NP, F), jnp.float32),
            pltpu.SemaphoreType.DMA,
            pltpu.SemaphoreType.DMA,
            pltpu.SemaphoreType.DMA,
            pltpu.SemaphoreType.DMA,
        ],
    )
    return fn(y, src_p, dst_p, zeros_blk)


def kernel(features, edge_index, W0, W1, W2, ThW, Thb, Th1W, Th1b, Th2W, Th2b):
    f_pad = jnp.pad(features, ((0, NP - N), (0, 0)))
    dst = edge_index[0]
    src = edge_index[1]
    pad_e = EP - E
    pad_src = (jnp.arange(pad_e, dtype=jnp.int32) * 37) % N
    pad_dst = N + (jnp.arange(pad_e, dtype=jnp.int32) % (NP - N))
    src_p = jnp.concatenate([src, pad_src]).reshape(NW, NSEG, SEGR, C)
    dst_p = jnp.concatenate([dst, pad_dst]).reshape(NW, NSEG, SEGR, C)
    zeros_blk = jnp.zeros((RPT, F), jnp.float32)

    A0, B0 = ThW[:F], ThW[F:]
    A1, B1 = Th1W[:F], Th1W[F:]
    A2, B2 = Th2W[:F], Th2W[F:]
    b0 = Thb.reshape(1, F)
    b1 = Th1b.reshape(1, F)
    b2 = Th2b.reshape(1, F)

    y = _mm(f_pad, W0)
    p = _spmm(y, src_p, dst_p, zeros_blk)
    y = _dense(p, p, f_pad, A0, B0, b0, W1)
    p = _spmm(y, src_p, dst_p, zeros_blk)
    y = _dense(p, p, f_pad, A1, B1, b1, W2)
    p = _spmm(y, src_p, dst_p, zeros_blk)
    emb = _dense_last(p, p, f_pad, A2, B2, b2)
    return emb[:N]


# speedup vs baseline: 10.6949x; 1.2005x over previous
import jax
import jax.numpy as jnp
from jax import lax
from jax.experimental import pallas as pl
from jax.experimental.pallas import tpu as pltpu
from jax.experimental.pallas import tpu_sc as plsc

N = 10000
F = 128
E = 320000

NP = 10240
NW = 32
C = 128
NSEG = 2
SEGR = 40
UNROLL = 10
SEGCH = SEGR
EPW = NSEG * SEGR * C
EP = NW * EPW
RPT = NP // 16
BLK = 512


def _dense_body(p0_ref, p1_ref, f_ref, a_ref, b_ref, bias_ref, w_ref, o_ref):
    h = p0_ref[...] + p1_ref[...]
    hid = jnp.maximum(
        jnp.dot(h, a_ref[...], preferred_element_type=jnp.float32)
        + jnp.dot(f_ref[...], b_ref[...], preferred_element_type=jnp.float32)
        + bias_ref[...],
        0.0,
    )
    o_ref[...] = jnp.dot(hid, w_ref[...], preferred_element_type=jnp.float32)


def _dense_last_body(p0_ref, p1_ref, f_ref, a_ref, b_ref, bias_ref, o_ref):
    h = p0_ref[...] + p1_ref[...]
    o_ref[...] = jnp.maximum(
        jnp.dot(h, a_ref[...], preferred_element_type=jnp.float32)
        + jnp.dot(f_ref[...], b_ref[...], preferred_element_type=jnp.float32)
        + bias_ref[...],
        0.0,
    )


def _mm_body(x_ref, w_ref, o_ref):
    o_ref[...] = jnp.dot(x_ref[...], w_ref[...], preferred_element_type=jnp.float32)


_row_spec = pl.BlockSpec((BLK, F), lambda i: (i, 0))
_w_spec = pl.BlockSpec((F, F), lambda i: (0, 0))
_bias_spec = pl.BlockSpec((1, F), lambda i: (0, 0))
_p1_spec = pl.BlockSpec((BLK, F), lambda i: (i + NP // BLK, 0))

_GRID = (NP // BLK,)
_OUT = jax.ShapeDtypeStruct((NP, F), jnp.float32)

_mm = pl.pallas_call(
    _mm_body,
    grid=_GRID,
    in_specs=[_row_spec, _w_spec],
    out_specs=_row_spec,
    out_shape=_OUT,
)

_dense = pl.pallas_call(
    _dense_body,
    grid=_GRID,
    in_specs=[_row_spec, _p1_spec, _row_spec, _w_spec, _w_spec, _bias_spec, _w_spec],
    out_specs=_row_spec,
    out_shape=_OUT,
)

_dense_last = pl.pallas_call(
    _dense_last_body,
    grid=_GRID,
    in_specs=[_row_spec, _p1_spec, _row_spec, _w_spec, _w_spec, _bias_spec],
    out_specs=_row_spec,
    out_shape=_OUT,
)


def _spmm_body(y_hbm, src_hbm, dst_hbm, zeros_hbm, out_hbm,
               src_v, dst_v, b0, b1, acc, g0, g1, s0, s1):
    cid = lax.axis_index("c")
    sid = lax.axis_index("s")
    w = cid * 16 + sid
    pltpu.sync_copy(zeros_hbm, acc.at[pl.ds(sid * RPT, RPT)])
    plsc.subcore_barrier()

    def gath(c, buf, sem):
        return pltpu.async_copy(y_hbm.at[src_v.at[c]], buf, sem)

    def scat(c, buf, sem):
        return pltpu.async_copy(buf, acc.at[dst_v.at[c]], sem, add=True)

    def seg_body(seg, carry):
        pltpu.sync_copy(src_hbm.at[w, seg], src_v)
        pltpu.sync_copy(dst_hbm.at[w, seg], dst_v)

        def body(t, carry):
            a = UNROLL * t
            bufs = (b0, b1)
            gsems = (g0, g1)
            ssems = (s0, s1)
            g = [gath(a, b0, g0), gath(a + 1, b1, g1)]
            s = [None, None]
            for k in range(UNROLL):
                p = k % 2
                g[p].wait()
                s[p] = scat(a + k, bufs[p], ssems[p])
                if k + 2 < UNROLL:
                    s[p].wait()
                    g[p] = gath(a + k + 2, bufs[p], gsems[p])
            s[0].wait()
            s[1].wait()
            return carry

        lax.fori_loop(0, SEGCH // UNROLL, body, 0)
        return carry

    lax.fori_loop(0, NSEG, seg_body, 0)
    plsc.subcore_barrier()
    pltpu.sync_copy(acc.at[pl.ds(sid * RPT, RPT)],
                    out_hbm.at[pl.ds(cid * NP + sid * RPT, RPT)])


def _spmm(y, src_p, dst_p, zeros_blk):
    fn = pl.kernel(
        _spmm_body,
        out_type=jax.ShapeDtypeStruct((2 * NP, F), jnp.float32),
        mesh=plsc.VectorSubcoreMesh(core_axis_name="c", subcore_axis_name="s"),
        scratch_types=[
            pltpu.VMEM((SEGCH, C), jnp.int32),
            pltpu.VMEM((SEGCH, C), jnp.int32),
            pltpu.VMEM((C, F), jnp.float32),
            pltpu.VMEM((C, F), jnp.float32),
            pltpu.VMEM_SHARED((NP, F), jnp.float32),
            pltpu.SemaphoreType.DMA,
            pltpu.SemaphoreType.DMA,
            pltpu.SemaphoreType.DMA,
            pltpu.SemaphoreType.DMA,
        ],
    )
    return fn(y, src_p, dst_p, zeros_blk)


def kernel(features, edge_index, W0, W1, W2, ThW, Thb, Th1W, Th1b, Th2W, Th2b):
    f_pad = jnp.pad(features, ((0, NP - N), (0, 0)))
    dst = edge_index[0]
    src = edge_index[1]
    pad_e = EP - E
    pad_src = (jnp.arange(pad_e, dtype=jnp.int32) * 37) % N
    pad_dst = N + (jnp.arange(pad_e, dtype=jnp.int32) % (NP - N))
    src_p = jnp.concatenate([src, pad_src]).reshape(NW, NSEG, SEGR, C)
    dst_p = jnp.concatenate([dst, pad_dst]).reshape(NW, NSEG, SEGR, C)
    zeros_blk = jnp.zeros((RPT, F), jnp.float32)

    A0, B0 = ThW[:F], ThW[F:]
    A1, B1 = Th1W[:F], Th1W[F:]
    A2, B2 = Th2W[:F], Th2W[F:]
    b0 = Thb.reshape(1, F)
    b1 = Th1b.reshape(1, F)
    b2 = Th2b.reshape(1, F)

    y = _mm(f_pad, W0)
    p = _spmm(y, src_p, dst_p, zeros_blk)
    y = _dense(p, p, f_pad, A0, B0, b0, W1)
    p = _spmm(y, src_p, dst_p, zeros_blk)
    y = _dense(p, p, f_pad, A1, B1, b1, W2)
    p = _spmm(y, src_p, dst_p, zeros_blk)
    emb = _dense_last(p, p, f_pad, A2, B2, b2)
    return emb[:N]


# speedup vs baseline: 11.0577x; 1.0339x over previous
import jax
import jax.numpy as jnp
from jax import lax
from jax.experimental import pallas as pl
from jax.experimental.pallas import tpu as pltpu
from jax.experimental.pallas import tpu_sc as plsc

N = 10000
F = 128
E = 320000

NP = 10240
NW = 32
C = 128
NSEG = 2
SEGR = 40
UNROLL = 20
SEGCH = SEGR
EPW = NSEG * SEGR * C
EP = NW * EPW
RPT = NP // 16
BLK = 512


def _dense_body(p0_ref, p1_ref, f_ref, a_ref, b_ref, bias_ref, w_ref, o_ref):
    h = p0_ref[...] + p1_ref[...]
    hid = jnp.maximum(
        jnp.dot(h, a_ref[...], preferred_element_type=jnp.float32)
        + jnp.dot(f_ref[...], b_ref[...], preferred_element_type=jnp.float32)
        + bias_ref[...],
        0.0,
    )
    o_ref[...] = jnp.dot(hid, w_ref[...], preferred_element_type=jnp.float32)


def _dense_last_body(p0_ref, p1_ref, f_ref, a_ref, b_ref, bias_ref, o_ref):
    h = p0_ref[...] + p1_ref[...]
    o_ref[...] = jnp.maximum(
        jnp.dot(h, a_ref[...], preferred_element_type=jnp.float32)
        + jnp.dot(f_ref[...], b_ref[...], preferred_element_type=jnp.float32)
        + bias_ref[...],
        0.0,
    )


def _mm_body(x_ref, w_ref, o_ref):
    o_ref[...] = jnp.dot(x_ref[...], w_ref[...], preferred_element_type=jnp.float32)


_row_spec = pl.BlockSpec((BLK, F), lambda i: (i, 0))
_w_spec = pl.BlockSpec((F, F), lambda i: (0, 0))
_bias_spec = pl.BlockSpec((1, F), lambda i: (0, 0))
_p1_spec = pl.BlockSpec((BLK, F), lambda i: (i + NP // BLK, 0))

_GRID = (NP // BLK,)
_OUT = jax.ShapeDtypeStruct((NP, F), jnp.float32)

_mm = pl.pallas_call(
    _mm_body,
    grid=_GRID,
    in_specs=[_row_spec, _w_spec],
    out_specs=_row_spec,
    out_shape=_OUT,
)

_dense = pl.pallas_call(
    _dense_body,
    grid=_GRID,
    in_specs=[_row_spec, _p1_spec, _row_spec, _w_spec, _w_spec, _bias_spec, _w_spec],
    out_specs=_row_spec,
    out_shape=_OUT,
)

_dense_last = pl.pallas_call(
    _dense_last_body,
    grid=_GRID,
    in_specs=[_row_spec, _p1_spec, _row_spec, _w_spec, _w_spec, _bias_spec],
    out_specs=_row_spec,
    out_shape=_OUT,
)


def _spmm_body(y_hbm, src_hbm, dst_hbm, zeros_hbm, out_hbm,
               src_v, dst_v, b0, b1, acc, g0, g1, s0, s1):
    cid = lax.axis_index("c")
    sid = lax.axis_index("s")
    w = cid * 16 + sid
    pltpu.sync_copy(zeros_hbm, acc.at[pl.ds(sid * RPT, RPT)])
    plsc.subcore_barrier()

    def gath(c, buf, sem):
        return pltpu.async_copy(y_hbm.at[src_v.at[c]], buf, sem)

    def scat(c, buf, sem):
        return pltpu.async_copy(buf, acc.at[dst_v.at[c]], sem, add=True)

    def seg_body(seg, carry):
        pltpu.sync_copy(src_hbm.at[w, seg], src_v)
        pltpu.sync_copy(dst_hbm.at[w, seg], dst_v)

        def body(t, carry):
            a = UNROLL * t
            bufs = (b0, b1)
            gsems = (g0, g1)
            ssems = (s0, s1)
            g = [gath(a, b0, g0), gath(a + 1, b1, g1)]
            s = [None, None]
            for k in range(UNROLL):
                p = k % 2
                g[p].wait()
                s[p] = scat(a + k, bufs[p], ssems[p])
                if k + 2 < UNROLL:
                    s[p].wait()
                    g[p] = gath(a + k + 2, bufs[p], gsems[p])
            s[0].wait()
            s[1].wait()
            return carry

        lax.fori_loop(0, SEGCH // UNROLL, body, 0)
        return carry

    lax.fori_loop(0, NSEG, seg_body, 0)
    plsc.subcore_barrier()
    pltpu.sync_copy(acc.at[pl.ds(sid * RPT, RPT)],
                    out_hbm.at[pl.ds(cid * NP + sid * RPT, RPT)])


def _spmm(y, src_p, dst_p, zeros_blk):
    fn = pl.kernel(
        _spmm_body,
        out_type=jax.ShapeDtypeStruct((2 * NP, F), jnp.float32),
        mesh=plsc.VectorSubcoreMesh(core_axis_name="c", subcore_axis_name="s"),
        scratch_types=[
            pltpu.VMEM((SEGCH, C), jnp.int32),
            pltpu.VMEM((SEGCH, C), jnp.int32),
            pltpu.VMEM((C, F), jnp.float32),
            pltpu.VMEM((C, F), jnp.float32),
            pltpu.VMEM_SHARED((NP, F), jnp.float32),
            pltpu.SemaphoreType.DMA,
            pltpu.SemaphoreType.DMA,
            pltpu.SemaphoreType.DMA,
            pltpu.SemaphoreType.DMA,
        ],
    )
    return fn(y, src_p, dst_p, zeros_blk)


def kernel(features, edge_index, W0, W1, W2, ThW, Thb, Th1W, Th1b, Th2W, Th2b):
    f_pad = jnp.pad(features, ((0, NP - N), (0, 0)))
    dst = edge_index[0]
    src = edge_index[1]
    pad_e = EP - E
    pad_src = (jnp.arange(pad_e, dtype=jnp.int32) * 37) % N
    pad_dst = N + (jnp.arange(pad_e, dtype=jnp.int32) % (NP - N))
    src_p = jnp.concatenate([src, pad_src]).reshape(NW, NSEG, SEGR, C)
    dst_p = jnp.concatenate([dst, pad_dst]).reshape(NW, NSEG, SEGR, C)
    zeros_blk = jnp.zeros((RPT, F), jnp.float32)

    A0, B0 = ThW[:F], ThW[F:]
    A1, B1 = Th1W[:F], Th1W[F:]
    A2, B2 = Th2W[:F], Th2W[F:]
    b0 = Thb.reshape(1, F)
    b1 = Th1b.reshape(1, F)
    b2 = Th2b.reshape(1, F)

    y = _mm(f_pad, W0)
    p = _spmm(y, src_p, dst_p, zeros_blk)
    y = _dense(p, p, f_pad, A0, B0, b0, W1)
    p = _spmm(y, src_p, dst_p, zeros_blk)
    y = _dense(p, p, f_pad, A1, B1, b1, W2)
    p = _spmm(y, src_p, dst_p, zeros_blk)
    emb = _dense_last(p, p, f_pad, A2, B2, b2)
    return emb[:N]


# speedup vs baseline: 11.4992x; 1.0399x over previous
import jax
import jax.numpy as jnp
from jax import lax
from jax.experimental import pallas as pl
from jax.experimental.pallas import tpu as pltpu
from jax.experimental.pallas import tpu_sc as plsc

N = 10000
F = 128
E = 320000

NP = 10240
NW = 32
C = 128
NSEG = 2
SEGR = 40
UNROLL = 20
SEGCH = SEGR
EPW = NSEG * SEGR * C
EP = NW * EPW
RPT = NP // 16
BLK = 512


def _dense_body(p0_ref, p1_ref, f_ref, wa_ref, b_ref, bias_ref, o_ref):
    h = p0_ref[...] + p1_ref[...]
    o_ref[...] = jnp.maximum(
        jnp.dot(h, wa_ref[0], preferred_element_type=jnp.float32)
        + jnp.dot(f_ref[...], b_ref[...], preferred_element_type=jnp.float32)
        + bias_ref[...],
        0.0,
    )


def _wfuse_body(w_ref, a_ref, o_ref):
    o_ref[0] = jnp.dot(w_ref[0], a_ref[0], preferred_element_type=jnp.float32)


_row_spec = pl.BlockSpec((BLK, F), lambda i: (i, 0))
_w_spec = pl.BlockSpec((F, F), lambda i: (0, 0))
_bias_spec = pl.BlockSpec((1, F), lambda i: (0, 0))
_p1_spec = pl.BlockSpec((BLK, F), lambda i: (i + NP // BLK, 0))

_GRID = (NP // BLK,)
_OUT = jax.ShapeDtypeStruct((NP, F), jnp.float32)

_w3_spec = pl.BlockSpec((1, F, F), lambda i: (i, 0, 0))

_wfuse = pl.pallas_call(
    _wfuse_body,
    grid=(3,),
    in_specs=[_w3_spec, _w3_spec],
    out_specs=_w3_spec,
    out_shape=jax.ShapeDtypeStruct((3, F, F), jnp.float32),
)


def _make_dense(layer):
    wa_spec = pl.BlockSpec((1, F, F), lambda i, l=layer: (l, 0, 0))
    return pl.pallas_call(
        _dense_body,
        grid=_GRID,
        in_specs=[_row_spec, _p1_spec, _row_spec, wa_spec, _w_spec, _bias_spec],
        out_specs=_row_spec,
        out_shape=_OUT,
    )


_dense0 = _make_dense(0)
_dense1 = _make_dense(1)
_dense2 = _make_dense(2)


def _spmm_body(y_hbm, src_hbm, dst_hbm, zeros_hbm, out_hbm,
               src_v, dst_v, b0, b1, acc, g0, g1, s0, s1):
    cid = lax.axis_index("c")
    sid = lax.axis_index("s")
    w = cid * 16 + sid
    pltpu.sync_copy(zeros_hbm, acc.at[pl.ds(sid * RPT, RPT)])
    plsc.subcore_barrier()

    def gath(c, buf, sem):
        return pltpu.async_copy(y_hbm.at[src_v.at[c]], buf, sem)

    def scat(c, buf, sem):
        return pltpu.async_copy(buf, acc.at[dst_v.at[c]], sem, add=True)

    def seg_body(seg, carry):
        pltpu.sync_copy(src_hbm.at[w, seg], src_v)
        pltpu.sync_copy(dst_hbm.at[w, seg], dst_v)

        def body(t, carry):
            a = UNROLL * t
            bufs = (b0, b1)
            gsems = (g0, g1)
            ssems = (s0, s1)
            g = [gath(a, b0, g0), gath(a + 1, b1, g1)]
            s = [None, None]
            for k in range(UNROLL):
                p = k % 2
                g[p].wait()
                s[p] = scat(a + k, bufs[p], ssems[p])
                if k + 2 < UNROLL:
                    s[p].wait()
                    g[p] = gath(a + k + 2, bufs[p], gsems[p])
            s[0].wait()
            s[1].wait()
            return carry

        lax.fori_loop(0, SEGCH // UNROLL, body, 0)
        return carry

    lax.fori_loop(0, NSEG, seg_body, 0)
    plsc.subcore_barrier()
    pltpu.sync_copy(acc.at[pl.ds(sid * RPT, RPT)],
                    out_hbm.at[pl.ds(cid * NP + sid * RPT, RPT)])


def _spmm(y, src_p, dst_p, zeros_blk):
    fn = pl.kernel(
        _spmm_body,
        out_type=jax.ShapeDtypeStruct((2 * NP, F), jnp.float32),
        mesh=plsc.VectorSubcoreMesh(core_axis_name="c", subcore_axis_name="s"),
        scratch_types=[
            pltpu.VMEM((SEGCH, C), jnp.int32),
            pltpu.VMEM((SEGCH, C), jnp.int32),
            pltpu.VMEM((C, F), jnp.float32),
            pltpu.VMEM((C, F), jnp.float32),
            pltpu.VMEM_SHARED((NP, F), jnp.float32),
            pltpu.SemaphoreType.DMA,
            pltpu.SemaphoreType.DMA,
            pltpu.SemaphoreType.DMA,
            pltpu.SemaphoreType.DMA,
        ],
    )
    return fn(y, src_p, dst_p, zeros_blk)


def kernel(features, edge_index, W0, W1, W2, ThW, Thb, Th1W, Th1b, Th2W, Th2b):
    f_pad = jnp.pad(features, ((0, NP - N), (0, 0)))
    dst = edge_index[0]
    src = edge_index[1]
    pad_e = EP - E
    pad_src = (jnp.arange(pad_e, dtype=jnp.int32) * 37) % N
    pad_dst = N + (jnp.arange(pad_e, dtype=jnp.int32) % (NP - N))
    src_p = jnp.concatenate([src, pad_src]).reshape(NW, NSEG, SEGR, C)
    dst_p = jnp.concatenate([dst, pad_dst]).reshape(NW, NSEG, SEGR, C)
    zeros_blk = jnp.zeros((RPT, F), jnp.float32)

    Ws = jnp.stack([W0, W1, W2])
    As = jnp.stack([ThW[:F], Th1W[:F], Th2W[:F]])
    WA = _wfuse(Ws, As)
    B0, B1, B2 = ThW[F:], Th1W[F:], Th2W[F:]
    b0 = Thb.reshape(1, F)
    b1 = Th1b.reshape(1, F)
    b2 = Th2b.reshape(1, F)

    p = _spmm(f_pad, src_p, dst_p, zeros_blk)
    h = _dense0(p, p, f_pad, WA, B0, b0)
    p = _spmm(h, src_p, dst_p, zeros_blk)
    h = _dense1(p, p, f_pad, WA, B1, b1)
    p = _spmm(h, src_p, dst_p, zeros_blk)
    emb = _dense2(p, p, f_pad, WA, B2, b2)
    return emb[:N]
